# Initial kernel scaffold; baseline (speedup 1.0000x reference)
import jax
import jax.numpy as jnp
from jax.experimental import pallas as pl


def kernel(input_pos, k_val, v_val, k_cache, v_cache):
    raise NotImplementedError("write your pallas kernel here")


# speedup vs baseline: 1.1364x; 1.1364x over previous
import functools

import jax
import jax.numpy as jnp
from jax import lax
from jax.experimental import pallas as pl
from jax.experimental.pallas import tpu as pltpu
from jax.experimental.pallas import tpu_sc as plsc

_B, _H, _S, _D = 8, 16, 2048, 128
_Q = 32
_BH = _B * _H
_R = _BH * _Q
_NC, _NS = 2, 16
_NW = _NC * _NS
_RPW = _R // _NW


@functools.partial(
    pl.kernel,
    out_type=(),
    mesh=plsc.VectorSubcoreMesh(core_axis_name="c", subcore_axis_name="s"),
    scratch_types=[
        pltpu.VMEM((_RPW,), jnp.int32),
        pltpu.VMEM((_RPW, _D), jnp.float32),
        pltpu.VMEM((_RPW, _D), jnp.float32),
        pltpu.SemaphoreType.DMA,
        pltpu.SemaphoreType.DMA,
    ],
)
def _scatter_rows(idx_hbm, kv_hbm, vv_hbm, ko_ref, vo_ref,
                  idx_v, krows_v, vrows_v, ksem, vsem):
    wid = lax.axis_index("s") * _NC + lax.axis_index("c")
    base = wid * _RPW
    pltpu.sync_copy(idx_hbm.at[pl.ds(base, _RPW)], idx_v)
    pltpu.sync_copy(kv_hbm.at[pl.ds(base, _RPW)], krows_v)
    pltpu.sync_copy(vv_hbm.at[pl.ds(base, _RPW)], vrows_v)
    kcopy = pltpu.async_copy(krows_v, ko_ref.at[idx_v], ksem)
    vcopy = pltpu.async_copy(vrows_v, vo_ref.at[idx_v], vsem)
    kcopy.wait()
    vcopy.wait()


def kernel(input_pos, k_val, v_val, k_cache, v_cache):
    pos = input_pos.astype(jnp.int32)
    idx = (jnp.arange(_BH, dtype=jnp.int32)[:, None] * _S
           + pos[None, :]).reshape(_R)
    kv = k_val.reshape(_R, _D)
    vv = v_val.reshape(_R, _D)
    ko = jax.new_ref(k_cache.reshape(_BH * _S, _D))
    vo = jax.new_ref(v_cache.reshape(_BH * _S, _D))
    _scatter_rows(idx, kv, vv, ko, vo)
    return (ko[...].reshape(_B, _H, _S, _D),
            vo[...].reshape(_B, _H, _S, _D))
